# baseline (speedup 1.0000x reference)
import functools

import jax
import jax.numpy as jnp
from jax import lax
from jax.experimental import pallas as pl
from jax.experimental.pallas import tpu as pltpu
from jax.experimental.pallas import tpu_sc as plsc

NC, NS = 2, 16
NW = NC * NS


def _gather_body(bpw, idx_hbm, table_hbm, out_hbm, idx_s, sem):
    wid = lax.axis_index("s") * NC + lax.axis_index("c")
    base = wid * bpw
    pltpu.sync_copy(idx_hbm.at[pl.ds(base, bpw)], idx_s)

    @pl.loop(0, bpw // 16)
    def _issue(i):
        vec = idx_s[pl.ds(i * 16, 16)]
        for j in range(16):
            pltpu.async_copy(
                table_hbm.at[pl.ds(vec[j], 1)],
                out_hbm.at[pl.ds(base + i * 16 + j, 1)],
                sem,
            )

    pltpu.make_async_copy(
        table_hbm.at[pl.ds(0, bpw)], out_hbm.at[pl.ds(base, bpw)], sem
    ).wait()


def _sc_gather(idx, table):
    batch = idx.shape[0]
    emb_dim = table.shape[1]
    bpw = batch // NW
    mesh = plsc.VectorSubcoreMesh(
        core_axis_name="c", subcore_axis_name="s", num_cores=NC, num_subcores=NS
    )
    grab = pl.kernel(
        functools.partial(_gather_body, bpw),
        out_type=jax.ShapeDtypeStruct((batch, emb_dim), jnp.float32),
        mesh=mesh,
        scratch_types=[
            pltpu.VMEM((bpw,), jnp.int32),
            pltpu.SemaphoreType.DMA,
        ],
    )
    return grab(idx, table)


def _mlp_body(emb_ref, w1_ref, b1_ref, w2_ref, b2_ref, out_ref):
    h = jnp.dot(emb_ref[...], w1_ref[...], preferred_element_type=jnp.float32)
    h = jnp.maximum(h + b1_ref[...], 0.0)
    out_ref[...] = (
        jnp.dot(h, w2_ref[...], preferred_element_type=jnp.float32) + b2_ref[...]
    )


def _tc_mlp(emb, W1, b1, W2, b2, block_b=2048):
    batch, emb_dim = emb.shape
    out_dim = W2.shape[1]
    grid = (batch // block_b,)
    return pl.pallas_call(
        _mlp_body,
        grid=grid,
        in_specs=[
            pl.BlockSpec((block_b, emb_dim), lambda i: (i, 0)),
            pl.BlockSpec((emb_dim, emb_dim), lambda i: (0, 0)),
            pl.BlockSpec((1, emb_dim), lambda i: (0, 0)),
            pl.BlockSpec((emb_dim, out_dim), lambda i: (0, 0)),
            pl.BlockSpec((1, out_dim), lambda i: (0, 0)),
        ],
        out_specs=pl.BlockSpec((block_b, out_dim), lambda i: (i, 0)),
        out_shape=jax.ShapeDtypeStruct((batch, out_dim), jnp.float32),
    )(emb, W1, b1.reshape(1, -1), W2, b2.reshape(1, -1))


def kernel(user_id, table, W1, b1, W2, b2):
    idx = user_id.astype(jnp.int32)
    emb = _sc_gather(idx, table)
    return _tc_mlp(emb, W1, b1, W2, b2)


# speedup vs baseline: 1.6436x; 1.6436x over previous
import functools

import jax
import jax.numpy as jnp
from jax import lax
from jax.experimental import pallas as pl
from jax.experimental.pallas import tpu as pltpu
from jax.experimental.pallas import tpu_sc as plsc

NC, NS = 2, 16
NW = NC * NS


def _gather_body(bpw, idx_hbm, table_hbm, out_hbm, idx_s, rows_v, sem):
    wid = lax.axis_index("s") * NC + lax.axis_index("c")
    base = wid * bpw
    pltpu.sync_copy(idx_hbm.at[pl.ds(base, bpw)], idx_s)

    @pl.loop(0, bpw // 16)
    def _issue(i):
        vec = idx_s[pl.ds(i * 16, 16)]
        for j in range(16):
            pltpu.async_copy(
                table_hbm.at[pl.ds(vec[j], 1)],
                rows_v.at[pl.ds(i * 16 + j, 1)],
                sem,
            )

    pltpu.make_async_copy(table_hbm.at[pl.ds(0, bpw)], rows_v, sem).wait()
    pltpu.sync_copy(rows_v, out_hbm.at[pl.ds(base, bpw)])


def _sc_gather(idx, table):
    batch = idx.shape[0]
    emb_dim = table.shape[1]
    bpw = batch // NW
    mesh = plsc.VectorSubcoreMesh(
        core_axis_name="c", subcore_axis_name="s", num_cores=NC, num_subcores=NS
    )
    grab = pl.kernel(
        functools.partial(_gather_body, bpw),
        out_type=jax.ShapeDtypeStruct((batch, emb_dim), jnp.float32),
        mesh=mesh,
        scratch_types=[
            pltpu.VMEM((bpw,), jnp.int32),
            pltpu.VMEM((bpw, emb_dim), jnp.float32),
            pltpu.SemaphoreType.DMA,
        ],
    )
    return grab(idx, table)


def _mlp_body(emb_ref, w1_ref, b1_ref, w2_ref, b2_ref, out_ref):
    h = jnp.dot(emb_ref[...], w1_ref[...], preferred_element_type=jnp.float32)
    h = jnp.maximum(h + b1_ref[...], 0.0)
    out_ref[...] = (
        jnp.dot(h, w2_ref[...], preferred_element_type=jnp.float32) + b2_ref[...]
    )


def _tc_mlp(emb, W1, b1, W2, b2, block_b=2048):
    batch, emb_dim = emb.shape
    out_dim = W2.shape[1]
    grid = (batch // block_b,)
    return pl.pallas_call(
        _mlp_body,
        grid=grid,
        in_specs=[
            pl.BlockSpec((block_b, emb_dim), lambda i: (i, 0)),
            pl.BlockSpec((emb_dim, emb_dim), lambda i: (0, 0)),
            pl.BlockSpec((1, emb_dim), lambda i: (0, 0)),
            pl.BlockSpec((emb_dim, out_dim), lambda i: (0, 0)),
            pl.BlockSpec((1, out_dim), lambda i: (0, 0)),
        ],
        out_specs=pl.BlockSpec((block_b, out_dim), lambda i: (i, 0)),
        out_shape=jax.ShapeDtypeStruct((batch, out_dim), jnp.float32),
    )(emb, W1, b1.reshape(1, -1), W2, b2.reshape(1, -1))


def kernel(user_id, table, W1, b1, W2, b2):
    idx = user_id.astype(jnp.int32)
    half = idx.shape[0] // 2
    emb0 = _sc_gather(idx[:half], table)
    emb1 = _sc_gather(idx[half:], table)
    out0 = _tc_mlp(emb0, W1, b1, W2, b2)
    out1 = _tc_mlp(emb1, W1, b1, W2, b2)
    return jnp.concatenate([out0, out1], axis=0)
